# baseline (speedup 1.0000x reference)
import functools

import jax
import jax.numpy as jnp
from jax import lax
from jax.experimental import pallas as pl
from jax.experimental.pallas import tpu as pltpu
from jax.experimental.pallas import tpu_sc as plsc

_NUM_CORES = 2
_NUM_SUBCORES = 16
_NUM_WORKERS = _NUM_CORES * _NUM_SUBCORES
_LANES = 16


@functools.lru_cache(maxsize=None)
def _build(B, V, D):
    assert B % (_NUM_WORKERS * _LANES) == 0 and V % 2 == 0
    b_per_w = B // _NUM_WORKERS
    n_chunks = b_per_w // _LANES
    mesh = plsc.VectorSubcoreMesh(core_axis_name="c", subcore_axis_name="s")

    @functools.partial(
        pl.kernel,
        mesh=mesh,
        out_type=jax.ShapeDtypeStruct((B // 2, 2 * D), jnp.float32),
        scratch_types=[
            pltpu.VMEM((b_per_w,), jnp.int32),
            pltpu.VMEM((b_per_w, 2 * D), jnp.float32),
            pltpu.SemaphoreType.DMA,
        ],
        compiler_params=pltpu.CompilerParams(needs_layout_passes=False),
    )
    def gather_kernel(msg_hbm, tbl_hbm, out_hbm, idx_v, lines_v, sem):
        wid = lax.axis_index("s") * _NUM_CORES + lax.axis_index("c")
        base = wid * b_per_w
        pltpu.sync_copy(msg_hbm.at[pl.ds(base, b_per_w)], idx_v)

        def fire_chunk(c, carry):
            idxvec = idx_v[pl.ds(c * _LANES, _LANES)]
            pairvec = lax.shift_right_logical(idxvec, 1)
            for u in range(_LANES):
                pltpu.async_copy(
                    tbl_hbm.at[pl.ds(pairvec[u], 1), 0, :],
                    lines_v.at[pl.ds(c * _LANES + u, 1), :],
                    sem,
                )
            return carry

        lax.fori_loop(0, n_chunks, fire_chunk, 0)

        def drain(k, carry):
            pltpu.make_async_copy(
                tbl_hbm.at[pl.ds(0, 1), 0, :],
                lines_v.at[pl.ds(0, 1), :],
                sem,
            ).wait()
            return carry

        lax.fori_loop(0, b_per_w, drain, 0)

        iota = lax.iota(jnp.int32, _LANES)
        parity = jnp.bitwise_and(iota, 1) * D
        halfiota = lax.shift_right_logical(iota, 1)

        def compact(c, carry):
            rvec = c * _LANES + iota
            dstrow = c * (_LANES // 2) + halfiota
            offv = jnp.bitwise_and(idx_v[pl.ds(c * _LANES, _LANES)], 1) * D
            for j in range(D):
                col = plsc.load_gather(lines_v, [rvec, offv + j])
                plsc.store_scatter(lines_v, [dstrow, parity + j], col)
            return carry

        lax.fori_loop(0, n_chunks, compact, 0)
        obase = pl.multiple_of(base // 2, 8)
        pltpu.sync_copy(
            lines_v.at[pl.ds(0, b_per_w // 2), :],
            out_hbm.at[pl.ds(obase, b_per_w // 2)],
        )

    return gather_kernel


def kernel(message, embedding_table):
    B, = message.shape
    V, D = embedding_table.shape
    tbl3 = embedding_table.reshape(V // 2, 1, 2 * D)
    out2 = _build(B, V, D)(message.astype(jnp.int32), tbl3)
    return out2.reshape(B, D)


# speedup vs baseline: 2.7179x; 2.7179x over previous
import functools

import jax
import jax.numpy as jnp
from jax import lax
from jax.experimental import pallas as pl
from jax.experimental.pallas import tpu as pltpu
from jax.experimental.pallas import tpu_sc as plsc

_NUM_CORES = 2
_NUM_SUBCORES = 16
_NUM_WORKERS = _NUM_CORES * _NUM_SUBCORES
_LANES = 16


@functools.lru_cache(maxsize=None)
def _build(B, V, D):
    assert B % (_NUM_WORKERS * _LANES) == 0 and V % 8 == 0
    b_per_w = B // _NUM_WORKERS
    n_chunks = b_per_w // _LANES
    mesh = plsc.VectorSubcoreMesh(core_axis_name="c", subcore_axis_name="s")

    @functools.partial(
        pl.kernel,
        mesh=mesh,
        out_type=jax.ShapeDtypeStruct((B, D), jnp.float32),
        scratch_types=[
            pltpu.VMEM((b_per_w,), jnp.int32),
            pltpu.VMEM((b_per_w, D), jnp.float32),
            pltpu.SemaphoreType.DMA,
        ],
    )
    def gather_kernel(msg_hbm, tbl_hbm, out_hbm, idx_v, rows_v, sem):
        wid = lax.axis_index("s") * _NUM_CORES + lax.axis_index("c")
        base = wid * b_per_w
        pltpu.sync_copy(msg_hbm.at[pl.ds(base, b_per_w)], idx_v)

        def fire_chunk(c, carry):
            idxvec = idx_v[pl.ds(c * _LANES, _LANES)]
            blkvec = lax.shift_right_logical(idxvec, 3)
            subvec = jnp.bitwise_and(idxvec, 7)
            for u in range(_LANES):
                pltpu.async_copy(
                    tbl_hbm.at[pl.ds(blkvec[u], 1), subvec[u], :],
                    rows_v.at[pl.ds(c * _LANES + u, 1), :],
                    sem,
                )
            return carry

        lax.fori_loop(0, n_chunks, fire_chunk, 0)

        def drain(k, carry):
            pltpu.make_async_copy(
                tbl_hbm.at[pl.ds(0, 1), 0, :],
                rows_v.at[pl.ds(0, 1), :],
                sem,
            ).wait()
            return carry

        lax.fori_loop(0, b_per_w, drain, 0)
        pltpu.sync_copy(rows_v, out_hbm.at[pl.ds(base, b_per_w)])

    return gather_kernel


def kernel(message, embedding_table):
    B, = message.shape
    V, D = embedding_table.shape
    tbl3 = embedding_table.reshape(V // 8, 8, D)
    return _build(B, V, D)(message.astype(jnp.int32), tbl3)
